# Initial kernel scaffold; baseline (speedup 1.0000x reference)
import jax
import jax.numpy as jnp
from jax.experimental import pallas as pl


def kernel(x, edge_index, edge_type, batch, table, w_rel1, w_root1, b1, w_rel2, w_root2, b2, lin_w, lin_b):
    raise NotImplementedError("write your pallas kernel here")


# speedup vs baseline: 9.7003x; 9.7003x over previous
import functools

import jax
import jax.numpy as jnp
from jax import lax
from jax.experimental import pallas as pl
from jax.experimental.pallas import tpu as pltpu
from jax.experimental.pallas import tpu_sc as plsc

N = 50000
E = 800000
V = 100000
D = 64
R = 3
C = 2
G = 64

NC = 2
NS = 16
NW = NC * NS

TPB = 512
NB = 98
NP = NB * TPB

EPAD = 819200
ECH = 1024
ICH = 128
EPT = EPAD // NW
EPS = EPAD // NS
XPAD = 57344
XPT = XPAD // NW
XHT = XPT // 2

CNT_SZ = 201088
ACC_ROWS = N + 512
HW_ROWS = NB * 6 * TPB

_MESH = plsc.VectorSubcoreMesh(
    core_axis_name="c", subcore_axis_name="s", num_cores=NC, num_subcores=NS
)
_SC_PARAMS = pltpu.CompilerParams(use_tc_tiling_on_sc=False)


def _fill_const_1d(ref, n, val, dtype):
    def body(i):
        ref[pl.ds(i * 16, 16)] = jnp.full((16,), val, dtype)
    pl.loop(0, n // 16)(body)


def _sc_prep_body(et3, dst3, src3, xp, table, s3, gidx3, hraw,
                  cnt_sh, tbuf, dbuf, sbuf, ibuf, gbuf, vbuf, scb,
                  ones, zbuf, xbuf, erows, sem):
    c = lax.axis_index("c")
    s_ax = lax.axis_index("s")
    wid = s_ax * NC + c

    _fill_const_1d(zbuf, ECH, 0.0, jnp.float32)
    _fill_const_1d(ones, ICH, 1.0, jnp.float32)
    zper = CNT_SZ // NS
    zbase = s_ax * zper
    for k in range(zper // ECH):
        pltpu.sync_copy(zbuf, cnt_sh.at[pl.ds(zbase + k * ECH, ECH)])
    rem = zper % ECH
    if rem:
        pltpu.sync_copy(zbuf.at[pl.ds(0, rem)],
                        cnt_sh.at[pl.ds(zbase + (zper // ECH) * ECH, rem)])
    plsc.subcore_barrier()

    def cnt_step(i):
        b8 = s_ax * (EPS // ICH) + i * (ECH // ICH)
        pltpu.sync_copy(et3.at[pl.ds(b8, ECH // ICH)], tbuf)
        pltpu.sync_copy(dst3.at[pl.ds(b8, ECH // ICH)], dbuf)
        for k in range(ECH // 16):
            r, col = k // 8, (k % 8) * 16
            t16 = tbuf[r, pl.ds(col, 16)]
            d16 = dbuf[r, pl.ds(col, 16)]
            ibuf[r, pl.ds(col, 16)] = t16 * N + d16
        for j in range(ECH // ICH):
            pltpu.sync_copy(ones, cnt_sh.at[ibuf.at[j]], add=True)
    pl.loop(0, EPS // ECH)(cnt_step)
    plsc.subcore_barrier()

    def scale_step(i):
        b8 = wid * (EPT // ICH) + i * (ECH // ICH)
        pltpu.sync_copy(et3.at[pl.ds(b8, ECH // ICH)], tbuf)
        pltpu.sync_copy(dst3.at[pl.ds(b8, ECH // ICH)], dbuf)
        pltpu.sync_copy(src3.at[pl.ds(b8, ECH // ICH)], sbuf)
        for k in range(ECH // 16):
            r, col = k // 8, (k % 8) * 16
            t16 = tbuf[r, pl.ds(col, 16)]
            d16 = dbuf[r, pl.ds(col, 16)]
            s16 = sbuf[r, pl.ds(col, 16)]
            ibuf[r, pl.ds(col, 16)] = t16 * N + d16
            r16 = jnp.minimum(t16, 2)
            gbuf[r, pl.ds(col, 16)] = (
                (s16 >> 9) * (6 * TPB) + r16 * TPB + (s16 & 511))
        descs = [pltpu.async_copy(cnt_sh.at[ibuf.at[j]],
                                  vbuf.at[pl.ds(j * ICH, ICH)], sem)
                 for j in range(ECH // ICH)]
        for d in descs:
            d.wait()
        for k in range(ECH // 16):
            scb[pl.ds(k * 16, 16)] = (
                1.0 / jnp.maximum(vbuf[pl.ds(k * 16, 16)], 1.0))
        pltpu.sync_copy(scb, s3.at[pl.ds(wid * EPT + i * ECH, ECH)])
        pltpu.sync_copy(gbuf, gidx3.at[0, pl.ds(b8, ECH // ICH)])
        for k in range(ECH // 16):
            r, col = k // 8, (k % 8) * 16
            gbuf[r, pl.ds(col, 16)] = gbuf[r, pl.ds(col, 16)] + 3 * TPB
        pltpu.sync_copy(gbuf, gidx3.at[1, pl.ds(b8, ECH // ICH)])
    pl.loop(0, EPT // ECH)(scale_step)

    pltpu.sync_copy(xp.at[pl.ds(wid * XPT, XPT)], xbuf)
    for h in range(2):
        descs = [
            pltpu.async_copy(
                table.at[xbuf.at[pl.ds(h * XHT + j * ICH, ICH)]],
                erows.at[pl.ds(j * ICH, ICH)], sem)
            for j in range(XHT // ICH)]
        for d in descs:
            d.wait()

        def mask16(e0):
            x16 = xbuf[pl.ds(h * XHT + e0 * 16, 16)]
            m16 = jnp.where(x16 != 0, 1.0, 0.0)
            for u in range(16):
                e = e0 * 16 + u
                for q in range(D // 16):
                    erows[e, pl.ds(q * 16, 16)] = (
                        erows[e, pl.ds(q * 16, 16)] * m16[u])
        pl.loop(0, XHT // 16)(mask16)
        pltpu.sync_copy(erows, hraw.at[pl.ds(wid * XPT + h * XHT, XHT)])


_sc_prep = pl.kernel(
    _sc_prep_body,
    out_type=(
        jax.ShapeDtypeStruct((EPAD,), jnp.float32),
        jax.ShapeDtypeStruct((2, EPAD // ICH, ICH), jnp.int32),
        jax.ShapeDtypeStruct((XPAD, D), jnp.float32),
    ),
    mesh=_MESH,
    scratch_types=[
        pltpu.VMEM_SHARED((CNT_SZ,), jnp.float32),
        pltpu.VMEM((ECH // ICH, ICH), jnp.int32),
        pltpu.VMEM((ECH // ICH, ICH), jnp.int32),
        pltpu.VMEM((ECH // ICH, ICH), jnp.int32),
        pltpu.VMEM((ECH // ICH, ICH), jnp.int32),
        pltpu.VMEM((ECH // ICH, ICH), jnp.int32),
        pltpu.VMEM((ECH,), jnp.float32),
        pltpu.VMEM((ECH,), jnp.float32),
        pltpu.VMEM((ICH,), jnp.float32),
        pltpu.VMEM((ECH,), jnp.float32),
        pltpu.VMEM((XPT,), jnp.int32),
        pltpu.VMEM((XHT, D), jnp.float32),
        pltpu.SemaphoreType.DMA,
    ],
    compiler_params=_SC_PARAMS,
)


def _sc_agg_body(hw, gidx3, s1, dst3, zsrc, out,
                 acc_sh, gix, dbuf, scb, rows, sem):
    c = lax.axis_index("c")
    s_ax = lax.axis_index("s")
    wid = s_ax * NC + c

    zper = ACC_ROWS // NS
    zbase = s_ax * zper
    for k in range(zper // TPB):
        pltpu.sync_copy(zsrc, acc_sh.at[pl.ds(zbase + k * TPB, TPB)])
    rem = zper % TPB
    if rem:
        pltpu.sync_copy(zsrc.at[pl.ds(0, rem)],
                        acc_sh.at[pl.ds(zbase + (zper // TPB) * TPB, rem)])
    plsc.subcore_barrier()

    def step(i):
        b8 = s_ax * (EPS // ICH) + i * (ECH // ICH)
        pltpu.sync_copy(gidx3.at[c, pl.ds(b8, ECH // ICH)], gix)
        pltpu.sync_copy(dst3.at[pl.ds(b8, ECH // ICH)], dbuf)
        pltpu.sync_copy(s1.at[pl.ds(s_ax * EPS + i * ECH, ECH)], scb)
        for h in range(2):
            descs = [pltpu.async_copy(hw.at[gix.at[h * 4 + j]],
                                      rows.at[pl.ds(j * ICH, ICH)], sem)
                     for j in range(4)]
            for d in descs:
                d.wait()

            def scale16(e0):
                sv = scb[pl.ds(h * 512 + e0 * 16, 16)]
                for u in range(16):
                    e = e0 * 16 + u
                    rows[e, pl.ds(0, 16)] = rows[e, pl.ds(0, 16)] * sv[u]
                    rows[e, pl.ds(16, 16)] = (
                        rows[e, pl.ds(16, 16)] * sv[u])
            pl.loop(0, 512 // 16)(scale16)

            for j in range(4):
                pltpu.sync_copy(rows.at[pl.ds(j * ICH, ICH)],
                                acc_sh.at[dbuf.at[h * 4 + j]], add=True)
    pl.loop(0, EPS // ECH)(step)
    plsc.subcore_barrier()

    wper = NP // NS
    pltpu.sync_copy(acc_sh.at[pl.ds(s_ax * wper, wper)],
                    out.at[c, pl.ds(s_ax * wper, wper)])


_sc_agg = pl.kernel(
    _sc_agg_body,
    out_type=jax.ShapeDtypeStruct((2, NP, 32), jnp.float32),
    mesh=_MESH,
    scratch_types=[
        pltpu.VMEM_SHARED((ACC_ROWS, 32), jnp.float32),
        pltpu.VMEM((ECH // ICH, ICH), jnp.int32),
        pltpu.VMEM((ECH // ICH, ICH), jnp.int32),
        pltpu.VMEM((ECH,), jnp.float32),
        pltpu.VMEM((512, 32), jnp.float32),
        pltpu.SemaphoreType.DMA,
    ],
    compiler_params=_SC_PARAMS,
)


def _transform_blocks(h, wrel, wroot, bias, hw6, root):
    root[...] = jnp.dot(h, wroot[...],
                        preferred_element_type=jnp.float32) + bias[...]
    w = wrel[...]
    for j in range(6):
        r, kk = j % 3, j // 3
        hw6[0, j] = jnp.dot(h, w[r][:, kk * 32:(kk + 1) * 32],
                            preferred_element_type=jnp.float32)


def _tc_layer1_body(hraw, wrel, wroot, bias, hw6, root):
    _transform_blocks(hraw[...], wrel, wroot, bias, hw6, root)


def _tc_layer2_body(rootp, a0, a1, wrel, wroot, bias, hw6, root):
    h = jnp.maximum(
        rootp[...] + jnp.concatenate([a0[0], a1[0]], axis=1), 0.0)
    _transform_blocks(h, wrel, wroot, bias, hw6, root)


def _tc_pool_body(rootp, a0, a1, batch3, lin_w, lin_b, out, sacc, cacc):
    i = pl.program_id(0)

    @pl.when(i == 0)
    def _init():
        sacc[...] = jnp.zeros((G, D), jnp.float32)
        cacc[...] = jnp.zeros((G, D), jnp.float32)

    h2 = jnp.maximum(
        rootp[...] + jnp.concatenate([a0[0], a1[0]], axis=1), 0.0)
    ohT = (lax.broadcasted_iota(jnp.int32, (G, TPB), 0) == batch3[0]
           ).astype(jnp.float32)
    sacc[...] += jnp.dot(ohT, h2, preferred_element_type=jnp.float32)
    cacc[...] += jnp.dot(ohT, jnp.ones((TPB, D), jnp.float32),
                         preferred_element_type=jnp.float32)

    @pl.when(i == NB - 1)
    def _fin():
        g = sacc[...] / jnp.maximum(cacc[...], 1.0)
        out[...] = jnp.dot(g, lin_w[...],
                           preferred_element_type=jnp.float32) + lin_b[...]


_HW6_SHAPE = jax.ShapeDtypeStruct((NB, 6, TPB, 32), jnp.float32)
_ROOT_SHAPE = jax.ShapeDtypeStruct((NP, D), jnp.float32)

_hw6_spec = pl.BlockSpec((1, 6, TPB, 32), lambda i: (i, 0, 0, 0))
_root_spec = pl.BlockSpec((TPB, D), lambda i: (i, 0))
_wrel_spec = pl.BlockSpec((R, D, D), lambda i: (0, 0, 0))
_wroot_spec = pl.BlockSpec((D, D), lambda i: (0, 0))
_bias_spec = pl.BlockSpec((1, D), lambda i: (0, 0))
_a0_spec = pl.BlockSpec((1, TPB, 32), lambda i: (0, i, 0))
_a1_spec = pl.BlockSpec((1, TPB, 32), lambda i: (1, i, 0))
_x3_spec = pl.BlockSpec((1, 1, TPB), lambda i: (i, 0, 0))

_tc_layer1 = pl.pallas_call(
    _tc_layer1_body,
    grid=(NB,),
    in_specs=[pl.BlockSpec((TPB, D), lambda i: (i, 0)),
              _wrel_spec, _wroot_spec, _bias_spec],
    out_specs=[_hw6_spec, _root_spec],
    out_shape=[_HW6_SHAPE, _ROOT_SHAPE],
)

_tc_layer2 = pl.pallas_call(
    _tc_layer2_body,
    grid=(NB,),
    in_specs=[_root_spec, _a0_spec, _a1_spec,
              _wrel_spec, _wroot_spec, _bias_spec],
    out_specs=[_hw6_spec, _root_spec],
    out_shape=[_HW6_SHAPE, _ROOT_SHAPE],
)

_tc_pool = pl.pallas_call(
    _tc_pool_body,
    grid=(NB,),
    in_specs=[_root_spec, _a0_spec, _a1_spec, _x3_spec,
              pl.BlockSpec((D, C), lambda i: (0, 0)),
              pl.BlockSpec((1, C), lambda i: (0, 0))],
    out_specs=pl.BlockSpec((G, C), lambda i: (0, 0)),
    out_shape=jax.ShapeDtypeStruct((G, C), jnp.float32),
    scratch_shapes=[pltpu.VMEM((G, D), jnp.float32),
                    pltpu.VMEM((G, D), jnp.float32)],
)


def kernel(x, edge_index, edge_type, batch, table, w_rel1, w_root1, b1,
           w_rel2, w_root2, b2, lin_w, lin_b):
    x = x.astype(jnp.int32)
    src = edge_index[0].astype(jnp.int32)
    dst = edge_index[1].astype(jnp.int32)
    et = edge_type.astype(jnp.int32)
    batch = batch.astype(jnp.int32)

    npad = EPAD - E
    j = jnp.arange(npad, dtype=jnp.int32)
    src_p = jnp.concatenate([src, j % 512])
    dst_p = jnp.concatenate([dst, N + (j % 512)])
    et_p = jnp.concatenate([et, jnp.full((npad,), R, jnp.int32)])
    xj = jnp.arange(XPAD - N, dtype=jnp.int32)
    x_p = jnp.concatenate([x, 1 + (xj % 512)])
    batch_p = jnp.concatenate(
        [batch, jnp.full((NP - N,), G, jnp.int32)]).reshape(NB, 1, TPB)

    et3 = et_p.reshape(EPAD // ICH, ICH)
    dst3 = dst_p.reshape(EPAD // ICH, ICH)
    src3 = src_p.reshape(EPAD // ICH, ICH)
    xp3 = x_p
    zsrc = jnp.zeros((TPB, 32), jnp.float32)

    s_e, gidx3, hraw = _sc_prep(et3, dst3, src3, xp3, table)

    hw1, root1 = _tc_layer1(hraw, w_rel1, w_root1, b1.reshape(1, D))
    agg1 = _sc_agg(hw1.reshape(HW_ROWS, 32), gidx3, s_e, dst3, zsrc)

    hw2, root2 = _tc_layer2(root1, agg1, agg1,
                            w_rel2, w_root2, b2.reshape(1, D))
    agg2 = _sc_agg(hw2.reshape(HW_ROWS, 32), gidx3, s_e, dst3, zsrc)

    return _tc_pool(root2, agg2, agg2, batch_p, lin_w, lin_b.reshape(1, C))


# speedup vs baseline: 10.5117x; 1.0836x over previous
import functools

import jax
import jax.numpy as jnp
from jax import lax
from jax.experimental import pallas as pl
from jax.experimental.pallas import tpu as pltpu
from jax.experimental.pallas import tpu_sc as plsc

N = 50000
E = 800000
V = 100000
D = 64
R = 3
C = 2
G = 64

NC = 2
NS = 16
NW = NC * NS

TPB = 512
NB = 98
NP = NB * TPB

EPAD = 819200
ECH = 1024
ICH = 128
EPT = EPAD // NW
EPS = EPAD // NS
XPAD = 57344
XPT = XPAD // NW
XHT = XPT // 2

CNT_SZ = 201088
ACC_ROWS = N + 512
HW_ROWS = NB * 6 * TPB

_MESH = plsc.VectorSubcoreMesh(
    core_axis_name="c", subcore_axis_name="s", num_cores=NC, num_subcores=NS
)
_SC_PARAMS = pltpu.CompilerParams(use_tc_tiling_on_sc=False)


def _fill_const_1d(ref, n, val, dtype):
    def body(i):
        ref[pl.ds(i * 16, 16)] = jnp.full((16,), val, dtype)
    pl.loop(0, n // 16)(body)


def _sc_prep_body(et3, dst3, src3, xp, table, s3, gidx3, hraw,
                  cnt_sh, tbuf, dbuf, sbuf, ibuf, gbuf, vbuf, scb,
                  ones, zbuf, xbuf, erows, sem):
    c = lax.axis_index("c")
    s_ax = lax.axis_index("s")
    wid = s_ax * NC + c

    _fill_const_1d(zbuf, ECH, 0.0, jnp.float32)
    _fill_const_1d(ones, ICH, 1.0, jnp.float32)
    zper = CNT_SZ // NS
    zbase = s_ax * zper
    for k in range(zper // ECH):
        pltpu.sync_copy(zbuf, cnt_sh.at[pl.ds(zbase + k * ECH, ECH)])
    rem = zper % ECH
    if rem:
        pltpu.sync_copy(zbuf.at[pl.ds(0, rem)],
                        cnt_sh.at[pl.ds(zbase + (zper // ECH) * ECH, rem)])
    plsc.subcore_barrier()

    def cnt_step(i):
        b8 = s_ax * (EPS // ICH) + i * (ECH // ICH)
        pltpu.sync_copy(et3.at[pl.ds(b8, ECH // ICH)], tbuf)
        pltpu.sync_copy(dst3.at[pl.ds(b8, ECH // ICH)], dbuf)
        for k in range(ECH // 16):
            r, col = k // 8, (k % 8) * 16
            t16 = tbuf[r, pl.ds(col, 16)]
            d16 = dbuf[r, pl.ds(col, 16)]
            ibuf[r, pl.ds(col, 16)] = t16 * N + d16
        for j in range(ECH // ICH):
            pltpu.sync_copy(ones, cnt_sh.at[ibuf.at[j]], add=True)
    pl.loop(0, EPS // ECH)(cnt_step)
    plsc.subcore_barrier()

    def scale_step(i):
        b8 = wid * (EPT // ICH) + i * (ECH // ICH)
        pltpu.sync_copy(et3.at[pl.ds(b8, ECH // ICH)], tbuf)
        pltpu.sync_copy(dst3.at[pl.ds(b8, ECH // ICH)], dbuf)
        pltpu.sync_copy(src3.at[pl.ds(b8, ECH // ICH)], sbuf)
        for k in range(ECH // 16):
            r, col = k // 8, (k % 8) * 16
            t16 = tbuf[r, pl.ds(col, 16)]
            d16 = dbuf[r, pl.ds(col, 16)]
            s16 = sbuf[r, pl.ds(col, 16)]
            ibuf[r, pl.ds(col, 16)] = t16 * N + d16
            r16 = jnp.minimum(t16, 2)
            gbuf[r, pl.ds(col, 16)] = (
                (s16 >> 9) * (6 * TPB) + r16 * TPB + (s16 & 511))
        descs = [pltpu.async_copy(cnt_sh.at[ibuf.at[j]],
                                  vbuf.at[pl.ds(j * ICH, ICH)], sem)
                 for j in range(ECH // ICH)]
        for d in descs:
            d.wait()
        for k in range(ECH // 16):
            scb[pl.ds(k * 16, 16)] = (
                1.0 / jnp.maximum(vbuf[pl.ds(k * 16, 16)], 1.0))
        pltpu.sync_copy(scb, s3.at[pl.ds(wid * EPT + i * ECH, ECH)])
        pltpu.sync_copy(gbuf, gidx3.at[0, pl.ds(b8, ECH // ICH)])
        for k in range(ECH // 16):
            r, col = k // 8, (k % 8) * 16
            gbuf[r, pl.ds(col, 16)] = gbuf[r, pl.ds(col, 16)] + 3 * TPB
        pltpu.sync_copy(gbuf, gidx3.at[1, pl.ds(b8, ECH // ICH)])
    pl.loop(0, EPT // ECH)(scale_step)

    pltpu.sync_copy(xp.at[pl.ds(wid * XPT, XPT)], xbuf)
    for h in range(2):
        descs = [
            pltpu.async_copy(
                table.at[xbuf.at[pl.ds(h * XHT + j * ICH, ICH)]],
                erows.at[pl.ds(j * ICH, ICH)], sem)
            for j in range(XHT // ICH)]
        for d in descs:
            d.wait()

        def mask16(e0):
            x16 = xbuf[pl.ds(h * XHT + e0 * 16, 16)]
            m16 = jnp.where(x16 != 0, 1.0, 0.0)
            for u in range(16):
                e = e0 * 16 + u
                for q in range(D // 16):
                    erows[e, pl.ds(q * 16, 16)] = (
                        erows[e, pl.ds(q * 16, 16)] * m16[u])
        pl.loop(0, XHT // 16)(mask16)
        pltpu.sync_copy(erows, hraw.at[pl.ds(wid * XPT + h * XHT, XHT)])


_sc_prep = pl.kernel(
    _sc_prep_body,
    out_type=(
        jax.ShapeDtypeStruct((EPAD,), jnp.float32),
        jax.ShapeDtypeStruct((2, EPAD // ICH, ICH), jnp.int32),
        jax.ShapeDtypeStruct((XPAD, D), jnp.float32),
    ),
    mesh=_MESH,
    scratch_types=[
        pltpu.VMEM_SHARED((CNT_SZ,), jnp.float32),
        pltpu.VMEM((ECH // ICH, ICH), jnp.int32),
        pltpu.VMEM((ECH // ICH, ICH), jnp.int32),
        pltpu.VMEM((ECH // ICH, ICH), jnp.int32),
        pltpu.VMEM((ECH // ICH, ICH), jnp.int32),
        pltpu.VMEM((ECH // ICH, ICH), jnp.int32),
        pltpu.VMEM((ECH,), jnp.float32),
        pltpu.VMEM((ECH,), jnp.float32),
        pltpu.VMEM((ICH,), jnp.float32),
        pltpu.VMEM((ECH,), jnp.float32),
        pltpu.VMEM((XPT,), jnp.int32),
        pltpu.VMEM((XHT, D), jnp.float32),
        pltpu.SemaphoreType.DMA,
    ],
    compiler_params=_SC_PARAMS,
)


def _sc_agg_body(hw, gidx3, s1, dst3, zsrc, out,
                 acc_sh, gix, dbuf, scb, rows0, rows1, sem):
    c = lax.axis_index("c")
    s_ax = lax.axis_index("s")
    wid = s_ax * NC + c
    rowsb = (rows0, rows1)

    zper = ACC_ROWS // NS
    zbase = s_ax * zper
    for k in range(zper // TPB):
        pltpu.sync_copy(zsrc, acc_sh.at[pl.ds(zbase + k * TPB, TPB)])
    rem = zper % TPB
    if rem:
        pltpu.sync_copy(zsrc.at[pl.ds(0, rem)],
                        acc_sh.at[pl.ds(zbase + (zper // TPB) * TPB, rem)])
    plsc.subcore_barrier()

    def step(i):
        b8 = s_ax * (EPS // ICH) + i * (ECH // ICH)
        pltpu.sync_copy(gidx3.at[c, pl.ds(b8, ECH // ICH)], gix)
        pltpu.sync_copy(dst3.at[pl.ds(b8, ECH // ICH)], dbuf)
        pltpu.sync_copy(s1.at[pl.ds(s_ax * EPS + i * ECH, ECH)], scb)

        def gathers(g):
            return [pltpu.async_copy(hw.at[gix.at[2 * g + j]],
                                     rowsb[g % 2].at[pl.ds(j * ICH, ICH)],
                                     sem)
                    for j in range(2)]

        descs = gathers(0)
        for g in range(4):
            nxt = gathers(g + 1) if g < 3 else None
            for d in descs:
                d.wait()
            rows = rowsb[g % 2]

            def scale16(e0, g=g, rows=rows):
                sv = scb[pl.ds(g * 256 + e0 * 16, 16)]
                for u in range(16):
                    e = e0 * 16 + u
                    rows[e, pl.ds(0, 16)] = rows[e, pl.ds(0, 16)] * sv[u]
                    rows[e, pl.ds(16, 16)] = (
                        rows[e, pl.ds(16, 16)] * sv[u])
            pl.loop(0, 256 // 16)(scale16)

            for j in range(2):
                pltpu.sync_copy(rows.at[pl.ds(j * ICH, ICH)],
                                acc_sh.at[dbuf.at[2 * g + j]], add=True)
            descs = nxt
    pl.loop(0, EPS // ECH)(step)
    plsc.subcore_barrier()

    wper = NP // NS
    pltpu.sync_copy(acc_sh.at[pl.ds(s_ax * wper, wper)],
                    out.at[c, pl.ds(s_ax * wper, wper)])


_sc_agg = pl.kernel(
    _sc_agg_body,
    out_type=jax.ShapeDtypeStruct((2, NP, 32), jnp.float32),
    mesh=_MESH,
    scratch_types=[
        pltpu.VMEM_SHARED((ACC_ROWS, 32), jnp.float32),
        pltpu.VMEM((ECH // ICH, ICH), jnp.int32),
        pltpu.VMEM((ECH // ICH, ICH), jnp.int32),
        pltpu.VMEM((ECH,), jnp.float32),
        pltpu.VMEM((256, 32), jnp.float32),
        pltpu.VMEM((256, 32), jnp.float32),
        pltpu.SemaphoreType.DMA,
    ],
    compiler_params=_SC_PARAMS,
)


def _transform_blocks(h, wrel, wroot, bias, hw6, root):
    root[...] = jnp.dot(h, wroot[...],
                        preferred_element_type=jnp.float32) + bias[...]
    w = wrel[...]
    for j in range(6):
        r, kk = j % 3, j // 3
        hw6[0, j] = jnp.dot(h, w[r][:, kk * 32:(kk + 1) * 32],
                            preferred_element_type=jnp.float32)


def _tc_layer1_body(hraw, wrel, wroot, bias, hw6, root):
    _transform_blocks(hraw[...], wrel, wroot, bias, hw6, root)


def _tc_layer2_body(rootp, a0, a1, wrel, wroot, bias, hw6, root):
    h = jnp.maximum(
        rootp[...] + jnp.concatenate([a0[0], a1[0]], axis=1), 0.0)
    _transform_blocks(h, wrel, wroot, bias, hw6, root)


def _tc_pool_body(rootp, a0, a1, batch3, lin_w, lin_b, out, sacc, cacc):
    i = pl.program_id(0)

    @pl.when(i == 0)
    def _init():
        sacc[...] = jnp.zeros((G, D), jnp.float32)
        cacc[...] = jnp.zeros((G, D), jnp.float32)

    h2 = jnp.maximum(
        rootp[...] + jnp.concatenate([a0[0], a1[0]], axis=1), 0.0)
    ohT = (lax.broadcasted_iota(jnp.int32, (G, TPB), 0) == batch3[0]
           ).astype(jnp.float32)
    sacc[...] += jnp.dot(ohT, h2, preferred_element_type=jnp.float32)
    cacc[...] += jnp.dot(ohT, jnp.ones((TPB, D), jnp.float32),
                         preferred_element_type=jnp.float32)

    @pl.when(i == NB - 1)
    def _fin():
        g = sacc[...] / jnp.maximum(cacc[...], 1.0)
        out[...] = jnp.dot(g, lin_w[...],
                           preferred_element_type=jnp.float32) + lin_b[...]


_HW6_SHAPE = jax.ShapeDtypeStruct((NB, 6, TPB, 32), jnp.float32)
_ROOT_SHAPE = jax.ShapeDtypeStruct((NP, D), jnp.float32)

_hw6_spec = pl.BlockSpec((1, 6, TPB, 32), lambda i: (i, 0, 0, 0))
_root_spec = pl.BlockSpec((TPB, D), lambda i: (i, 0))
_wrel_spec = pl.BlockSpec((R, D, D), lambda i: (0, 0, 0))
_wroot_spec = pl.BlockSpec((D, D), lambda i: (0, 0))
_bias_spec = pl.BlockSpec((1, D), lambda i: (0, 0))
_a0_spec = pl.BlockSpec((1, TPB, 32), lambda i: (0, i, 0))
_a1_spec = pl.BlockSpec((1, TPB, 32), lambda i: (1, i, 0))
_x3_spec = pl.BlockSpec((1, 1, TPB), lambda i: (i, 0, 0))

_tc_layer1 = pl.pallas_call(
    _tc_layer1_body,
    grid=(NB,),
    in_specs=[pl.BlockSpec((TPB, D), lambda i: (i, 0)),
              _wrel_spec, _wroot_spec, _bias_spec],
    out_specs=[_hw6_spec, _root_spec],
    out_shape=[_HW6_SHAPE, _ROOT_SHAPE],
)

_tc_layer2 = pl.pallas_call(
    _tc_layer2_body,
    grid=(NB,),
    in_specs=[_root_spec, _a0_spec, _a1_spec,
              _wrel_spec, _wroot_spec, _bias_spec],
    out_specs=[_hw6_spec, _root_spec],
    out_shape=[_HW6_SHAPE, _ROOT_SHAPE],
)

_tc_pool = pl.pallas_call(
    _tc_pool_body,
    grid=(NB,),
    in_specs=[_root_spec, _a0_spec, _a1_spec, _x3_spec,
              pl.BlockSpec((D, C), lambda i: (0, 0)),
              pl.BlockSpec((1, C), lambda i: (0, 0))],
    out_specs=pl.BlockSpec((G, C), lambda i: (0, 0)),
    out_shape=jax.ShapeDtypeStruct((G, C), jnp.float32),
    scratch_shapes=[pltpu.VMEM((G, D), jnp.float32),
                    pltpu.VMEM((G, D), jnp.float32)],
)


def kernel(x, edge_index, edge_type, batch, table, w_rel1, w_root1, b1,
           w_rel2, w_root2, b2, lin_w, lin_b):
    x = x.astype(jnp.int32)
    src = edge_index[0].astype(jnp.int32)
    dst = edge_index[1].astype(jnp.int32)
    et = edge_type.astype(jnp.int32)
    batch = batch.astype(jnp.int32)

    npad = EPAD - E
    j = jnp.arange(npad, dtype=jnp.int32)
    src_p = jnp.concatenate([src, j % 512])
    dst_p = jnp.concatenate([dst, N + (j % 512)])
    et_p = jnp.concatenate([et, jnp.full((npad,), R, jnp.int32)])
    xj = jnp.arange(XPAD - N, dtype=jnp.int32)
    x_p = jnp.concatenate([x, 1 + (xj % 512)])
    batch_p = jnp.concatenate(
        [batch, jnp.full((NP - N,), G, jnp.int32)]).reshape(NB, 1, TPB)

    et3 = et_p.reshape(EPAD // ICH, ICH)
    dst3 = dst_p.reshape(EPAD // ICH, ICH)
    src3 = src_p.reshape(EPAD // ICH, ICH)
    xp3 = x_p
    zsrc = jnp.zeros((TPB, 32), jnp.float32)

    s_e, gidx3, hraw = _sc_prep(et3, dst3, src3, xp3, table)

    hw1, root1 = _tc_layer1(hraw, w_rel1, w_root1, b1.reshape(1, D))
    agg1 = _sc_agg(hw1.reshape(HW_ROWS, 32), gidx3, s_e, dst3, zsrc)

    hw2, root2 = _tc_layer2(root1, agg1, agg1,
                            w_rel2, w_root2, b2.reshape(1, D))
    agg2 = _sc_agg(hw2.reshape(HW_ROWS, 32), gidx3, s_e, dst3, zsrc)

    return _tc_pool(root2, agg2, agg2, batch_p, lin_w, lin_b.reshape(1, C))


# speedup vs baseline: 10.6485x; 1.0130x over previous
import functools

import jax
import jax.numpy as jnp
from jax import lax
from jax.experimental import pallas as pl
from jax.experimental.pallas import tpu as pltpu
from jax.experimental.pallas import tpu_sc as plsc

N = 50000
E = 800000
V = 100000
D = 64
R = 3
C = 2
G = 64

NC = 2
NS = 16
NW = NC * NS

TPB = 512
NB = 98
NP = NB * TPB

EPAD = 819200
ECH = 1024
ICH = 128
EPT = EPAD // NW
EPS = EPAD // NS
XPAD = 57344
XPT = XPAD // NW
XHT = XPT // 2

CNT_SZ = 201088
ACC_ROWS = N + 512
HW_ROWS = NB * 6 * TPB

_MESH = plsc.VectorSubcoreMesh(
    core_axis_name="c", subcore_axis_name="s", num_cores=NC, num_subcores=NS
)
_SC_PARAMS = pltpu.CompilerParams(use_tc_tiling_on_sc=False)


def _fill_const_1d(ref, n, val, dtype):
    def body(i):
        ref[pl.ds(i * 16, 16)] = jnp.full((16,), val, dtype)
    pl.loop(0, n // 16)(body)


def _sc_prep_body(et3, dst3, src3, xp, table, s3, gidx3, hraw,
                  cnt_sh, tbuf, dbuf, sbuf, ibuf, gbuf, vbuf, scb,
                  ones, zbuf, xbuf, erows, sem):
    c = lax.axis_index("c")
    s_ax = lax.axis_index("s")
    wid = s_ax * NC + c

    _fill_const_1d(zbuf, ECH, 0.0, jnp.float32)
    _fill_const_1d(ones, ICH, 1.0, jnp.float32)
    zper = CNT_SZ // NS
    zbase = s_ax * zper
    for k in range(zper // ECH):
        pltpu.sync_copy(zbuf, cnt_sh.at[pl.ds(zbase + k * ECH, ECH)])
    rem = zper % ECH
    if rem:
        pltpu.sync_copy(zbuf.at[pl.ds(0, rem)],
                        cnt_sh.at[pl.ds(zbase + (zper // ECH) * ECH, rem)])
    plsc.subcore_barrier()

    def cnt_step(i):
        b8 = s_ax * (EPS // ICH) + i * (ECH // ICH)
        pltpu.sync_copy(et3.at[pl.ds(b8, ECH // ICH)], tbuf)
        pltpu.sync_copy(dst3.at[pl.ds(b8, ECH // ICH)], dbuf)
        for k in range(ECH // 16):
            r, col = k // 8, (k % 8) * 16
            t16 = tbuf[r, pl.ds(col, 16)]
            d16 = dbuf[r, pl.ds(col, 16)]
            ibuf[r, pl.ds(col, 16)] = t16 * N + d16
        for j in range(ECH // ICH):
            pltpu.sync_copy(ones, cnt_sh.at[ibuf.at[j]], add=True)
    pl.loop(0, EPS // ECH)(cnt_step)
    plsc.subcore_barrier()

    def scale_step(i):
        b8 = wid * (EPT // ICH) + i * (ECH // ICH)
        pltpu.sync_copy(et3.at[pl.ds(b8, ECH // ICH)], tbuf)
        pltpu.sync_copy(dst3.at[pl.ds(b8, ECH // ICH)], dbuf)
        pltpu.sync_copy(src3.at[pl.ds(b8, ECH // ICH)], sbuf)
        for k in range(ECH // 16):
            r, col = k // 8, (k % 8) * 16
            t16 = tbuf[r, pl.ds(col, 16)]
            d16 = dbuf[r, pl.ds(col, 16)]
            s16 = sbuf[r, pl.ds(col, 16)]
            ibuf[r, pl.ds(col, 16)] = t16 * N + d16
            r16 = jnp.minimum(t16, 2)
            gbuf[r, pl.ds(col, 16)] = (
                (s16 >> 9) * (6 * TPB) + r16 * TPB + (s16 & 511))
        descs = [pltpu.async_copy(cnt_sh.at[ibuf.at[j]],
                                  vbuf.at[pl.ds(j * ICH, ICH)], sem)
                 for j in range(ECH // ICH)]
        for d in descs:
            d.wait()
        for k in range(ECH // 16):
            scb[pl.ds(k * 16, 16)] = (
                1.0 / jnp.maximum(vbuf[pl.ds(k * 16, 16)], 1.0))
        pltpu.sync_copy(scb, s3.at[pl.ds(wid * EPT + i * ECH, ECH)])
        pltpu.sync_copy(gbuf, gidx3.at[0, pl.ds(b8, ECH // ICH)])
        for k in range(ECH // 16):
            r, col = k // 8, (k % 8) * 16
            gbuf[r, pl.ds(col, 16)] = gbuf[r, pl.ds(col, 16)] + 3 * TPB
        pltpu.sync_copy(gbuf, gidx3.at[1, pl.ds(b8, ECH // ICH)])
    pl.loop(0, EPT // ECH)(scale_step)

    pltpu.sync_copy(xp.at[pl.ds(wid * XPT, XPT)], xbuf)
    for h in range(2):
        descs = [
            pltpu.async_copy(
                table.at[xbuf.at[pl.ds(h * XHT + j * ICH, ICH)]],
                erows.at[pl.ds(j * ICH, ICH)], sem)
            for j in range(XHT // ICH)]
        for d in descs:
            d.wait()

        def mask16(e0):
            x16 = xbuf[pl.ds(h * XHT + e0 * 16, 16)]
            m16 = jnp.where(x16 != 0, 1.0, 0.0)
            for u in range(16):
                e = e0 * 16 + u
                for q in range(D // 16):
                    erows[e, pl.ds(q * 16, 16)] = (
                        erows[e, pl.ds(q * 16, 16)] * m16[u])
        pl.loop(0, XHT // 16)(mask16)
        pltpu.sync_copy(erows, hraw.at[pl.ds(wid * XPT + h * XHT, XHT)])


_sc_prep = pl.kernel(
    _sc_prep_body,
    out_type=(
        jax.ShapeDtypeStruct((EPAD,), jnp.float32),
        jax.ShapeDtypeStruct((2, EPAD // ICH, ICH), jnp.int32),
        jax.ShapeDtypeStruct((XPAD, D), jnp.float32),
    ),
    mesh=_MESH,
    scratch_types=[
        pltpu.VMEM_SHARED((CNT_SZ,), jnp.float32),
        pltpu.VMEM((ECH // ICH, ICH), jnp.int32),
        pltpu.VMEM((ECH // ICH, ICH), jnp.int32),
        pltpu.VMEM((ECH // ICH, ICH), jnp.int32),
        pltpu.VMEM((ECH // ICH, ICH), jnp.int32),
        pltpu.VMEM((ECH // ICH, ICH), jnp.int32),
        pltpu.VMEM((ECH,), jnp.float32),
        pltpu.VMEM((ECH,), jnp.float32),
        pltpu.VMEM((ICH,), jnp.float32),
        pltpu.VMEM((ECH,), jnp.float32),
        pltpu.VMEM((XPT,), jnp.int32),
        pltpu.VMEM((XHT, D), jnp.float32),
        pltpu.SemaphoreType.DMA,
    ],
    compiler_params=_SC_PARAMS,
)


def _sc_agg_body(hw, gidx3, s1, dst3, zsrc, out,
                 acc_sh, gix, dbuf, scb, rows0, rows1, sem, sem2):
    c = lax.axis_index("c")
    s_ax = lax.axis_index("s")
    wid = s_ax * NC + c
    rowsb = (rows0, rows1)

    zper = ACC_ROWS // NS
    zbase = s_ax * zper
    for k in range(zper // TPB):
        pltpu.sync_copy(zsrc, acc_sh.at[pl.ds(zbase + k * TPB, TPB)])
    rem = zper % TPB
    if rem:
        pltpu.sync_copy(zsrc.at[pl.ds(0, rem)],
                        acc_sh.at[pl.ds(zbase + (zper // TPB) * TPB, rem)])
    plsc.subcore_barrier()

    def step(i):
        b8 = s_ax * (EPS // ICH) + i * (ECH // ICH)
        pltpu.sync_copy(gidx3.at[c, pl.ds(b8, ECH // ICH)], gix)
        pltpu.sync_copy(dst3.at[pl.ds(b8, ECH // ICH)], dbuf)
        pltpu.sync_copy(s1.at[pl.ds(s_ax * EPS + i * ECH, ECH)], scb)

        def gathers(g):
            return [pltpu.async_copy(hw.at[gix.at[2 * g + j]],
                                     rowsb[g % 2].at[pl.ds(j * ICH, ICH)],
                                     sem)
                    for j in range(2)]

        descs = gathers(0)
        pend = [[], []]
        for g in range(4):
            if g < 3:
                for d in pend[(g + 1) % 2]:
                    d.wait()
                nxt = gathers(g + 1)
            else:
                nxt = None
            for d in descs:
                d.wait()
            rows = rowsb[g % 2]

            def scale16(e0, g=g, rows=rows):
                sv = scb[pl.ds(g * 256 + e0 * 16, 16)]
                for u in range(16):
                    e = e0 * 16 + u
                    rows[e, pl.ds(0, 16)] = rows[e, pl.ds(0, 16)] * sv[u]
                    rows[e, pl.ds(16, 16)] = (
                        rows[e, pl.ds(16, 16)] * sv[u])
            pl.loop(0, 256 // 16)(scale16)

            pend[g % 2] = [
                pltpu.async_copy(rows.at[pl.ds(j * ICH, ICH)],
                                 acc_sh.at[dbuf.at[2 * g + j]], sem2,
                                 add=True)
                for j in range(2)]
            descs = nxt
        for pp in pend:
            for d in pp:
                d.wait()
    pl.loop(0, EPS // ECH)(step)
    plsc.subcore_barrier()

    wper = NP // NS
    pltpu.sync_copy(acc_sh.at[pl.ds(s_ax * wper, wper)],
                    out.at[c, pl.ds(s_ax * wper, wper)])


_sc_agg = pl.kernel(
    _sc_agg_body,
    out_type=jax.ShapeDtypeStruct((2, NP, 32), jnp.float32),
    mesh=_MESH,
    scratch_types=[
        pltpu.VMEM_SHARED((ACC_ROWS, 32), jnp.float32),
        pltpu.VMEM((ECH // ICH, ICH), jnp.int32),
        pltpu.VMEM((ECH // ICH, ICH), jnp.int32),
        pltpu.VMEM((ECH,), jnp.float32),
        pltpu.VMEM((256, 32), jnp.float32),
        pltpu.VMEM((256, 32), jnp.float32),
        pltpu.SemaphoreType.DMA,
        pltpu.SemaphoreType.DMA,
    ],
    compiler_params=_SC_PARAMS,
)


def _transform_blocks(h, wrel, wroot, bias, hw6, root):
    root[...] = jnp.dot(h, wroot[...],
                        preferred_element_type=jnp.float32) + bias[...]
    w = wrel[...]
    for j in range(6):
        r, kk = j % 3, j // 3
        hw6[0, j] = jnp.dot(h, w[r][:, kk * 32:(kk + 1) * 32],
                            preferred_element_type=jnp.float32)


def _tc_layer1_body(hraw, wrel, wroot, bias, hw6, root):
    _transform_blocks(hraw[...], wrel, wroot, bias, hw6, root)


def _tc_layer2_body(rootp, a0, a1, wrel, wroot, bias, hw6, root):
    h = jnp.maximum(
        rootp[...] + jnp.concatenate([a0[0], a1[0]], axis=1), 0.0)
    _transform_blocks(h, wrel, wroot, bias, hw6, root)


def _tc_pool_body(rootp, a0, a1, batch3, lin_w, lin_b, out, sacc, cacc):
    i = pl.program_id(0)

    @pl.when(i == 0)
    def _init():
        sacc[...] = jnp.zeros((G, D), jnp.float32)
        cacc[...] = jnp.zeros((G, D), jnp.float32)

    h2 = jnp.maximum(
        rootp[...] + jnp.concatenate([a0[0], a1[0]], axis=1), 0.0)
    ohT = (lax.broadcasted_iota(jnp.int32, (G, TPB), 0) == batch3[0]
           ).astype(jnp.float32)
    sacc[...] += jnp.dot(ohT, h2, preferred_element_type=jnp.float32)
    cacc[...] += jnp.dot(ohT, jnp.ones((TPB, D), jnp.float32),
                         preferred_element_type=jnp.float32)

    @pl.when(i == NB - 1)
    def _fin():
        g = sacc[...] / jnp.maximum(cacc[...], 1.0)
        out[...] = jnp.dot(g, lin_w[...],
                           preferred_element_type=jnp.float32) + lin_b[...]


_HW6_SHAPE = jax.ShapeDtypeStruct((NB, 6, TPB, 32), jnp.float32)
_ROOT_SHAPE = jax.ShapeDtypeStruct((NP, D), jnp.float32)

_hw6_spec = pl.BlockSpec((1, 6, TPB, 32), lambda i: (i, 0, 0, 0))
_root_spec = pl.BlockSpec((TPB, D), lambda i: (i, 0))
_wrel_spec = pl.BlockSpec((R, D, D), lambda i: (0, 0, 0))
_wroot_spec = pl.BlockSpec((D, D), lambda i: (0, 0))
_bias_spec = pl.BlockSpec((1, D), lambda i: (0, 0))
_a0_spec = pl.BlockSpec((1, TPB, 32), lambda i: (0, i, 0))
_a1_spec = pl.BlockSpec((1, TPB, 32), lambda i: (1, i, 0))
_x3_spec = pl.BlockSpec((1, 1, TPB), lambda i: (i, 0, 0))

_tc_layer1 = pl.pallas_call(
    _tc_layer1_body,
    grid=(NB,),
    in_specs=[pl.BlockSpec((TPB, D), lambda i: (i, 0)),
              _wrel_spec, _wroot_spec, _bias_spec],
    out_specs=[_hw6_spec, _root_spec],
    out_shape=[_HW6_SHAPE, _ROOT_SHAPE],
)

_tc_layer2 = pl.pallas_call(
    _tc_layer2_body,
    grid=(NB,),
    in_specs=[_root_spec, _a0_spec, _a1_spec,
              _wrel_spec, _wroot_spec, _bias_spec],
    out_specs=[_hw6_spec, _root_spec],
    out_shape=[_HW6_SHAPE, _ROOT_SHAPE],
)

_tc_pool = pl.pallas_call(
    _tc_pool_body,
    grid=(NB,),
    in_specs=[_root_spec, _a0_spec, _a1_spec, _x3_spec,
              pl.BlockSpec((D, C), lambda i: (0, 0)),
              pl.BlockSpec((1, C), lambda i: (0, 0))],
    out_specs=pl.BlockSpec((G, C), lambda i: (0, 0)),
    out_shape=jax.ShapeDtypeStruct((G, C), jnp.float32),
    scratch_shapes=[pltpu.VMEM((G, D), jnp.float32),
                    pltpu.VMEM((G, D), jnp.float32)],
)


def kernel(x, edge_index, edge_type, batch, table, w_rel1, w_root1, b1,
           w_rel2, w_root2, b2, lin_w, lin_b):
    x = x.astype(jnp.int32)
    src = edge_index[0].astype(jnp.int32)
    dst = edge_index[1].astype(jnp.int32)
    et = edge_type.astype(jnp.int32)
    batch = batch.astype(jnp.int32)

    npad = EPAD - E
    j = jnp.arange(npad, dtype=jnp.int32)
    src_p = jnp.concatenate([src, j % 512])
    dst_p = jnp.concatenate([dst, N + (j % 512)])
    et_p = jnp.concatenate([et, jnp.full((npad,), R, jnp.int32)])
    xj = jnp.arange(XPAD - N, dtype=jnp.int32)
    x_p = jnp.concatenate([x, 1 + (xj % 512)])
    batch_p = jnp.concatenate(
        [batch, jnp.full((NP - N,), G, jnp.int32)]).reshape(NB, 1, TPB)

    et3 = et_p.reshape(EPAD // ICH, ICH)
    dst3 = dst_p.reshape(EPAD // ICH, ICH)
    src3 = src_p.reshape(EPAD // ICH, ICH)
    xp3 = x_p
    zsrc = jnp.zeros((TPB, 32), jnp.float32)

    s_e, gidx3, hraw = _sc_prep(et3, dst3, src3, xp3, table)

    hw1, root1 = _tc_layer1(hraw, w_rel1, w_root1, b1.reshape(1, D))
    agg1 = _sc_agg(hw1.reshape(HW_ROWS, 32), gidx3, s_e, dst3, zsrc)

    hw2, root2 = _tc_layer2(root1, agg1, agg1,
                            w_rel2, w_root2, b2.reshape(1, D))
    agg2 = _sc_agg(hw2.reshape(HW_ROWS, 32), gidx3, s_e, dst3, zsrc)

    return _tc_pool(root2, agg2, agg2, batch_p, lin_w, lin_b.reshape(1, C))
